# baseline (speedup 1.0000x reference)
import dataclasses
import functools

import jax
import jax.numpy as jnp
from jax import lax
from jax.experimental import pallas as pl
from jax.experimental.pallas import tpu as pltpu
from jax.experimental.pallas import tpu_sc as plsc

NC = 2
NS = 16
LN = 16
CH = 128


def _fast_rsqrt(d):
    ii = plsc.bitcast(d, jnp.int32)
    ii = jnp.int32(0x5F3759DF) - lax.shift_right_arithmetic(ii, 1)
    y = plsc.bitcast(ii, jnp.float32)
    half_d = d * jnp.float32(0.5)
    for _ in range(3):
        y = y * (jnp.float32(1.5) - half_d * y * y)
    return jnp.where(d > jnp.float32(0.0), y, jnp.float32(0.0))


def kernel(x, edge_index, edge_weights, W, b):
    N, D = x.shape
    E = edge_index.shape[1]
    NW = NC * NS
    npad_unit = NS * CH
    Npad = ((N + 1 + npad_unit - 1) // npad_unit) * npad_unit
    STRIPE = Npad // NS
    G = 8
    EPW = ((E + N + NW * G * CH - 1) // (NW * G * CH)) * (G * CH)
    NCHK = EPW // CH
    PADE = EPW * NW - (E + N)

    loop_idx = jnp.arange(N, dtype=jnp.int32)
    pad_ar = jnp.arange(PADE, dtype=jnp.int32)
    pad_row = pad_ar % N
    pad_col = N + (pad_ar * 7) % (Npad - N)
    one_b = jnp.int32(0x3F800000)
    cwr = jnp.concatenate([
        edge_index[1].astype(jnp.int32), loop_idx, pad_col,
        lax.bitcast_convert_type(edge_weights.astype(jnp.float32),
                                 jnp.int32),
        jnp.full((N,), one_b, jnp.int32), jnp.zeros((PADE,), jnp.int32),
    ]).reshape(2, NW, NCHK, CH)
    rowr = jnp.concatenate(
        [edge_index[0].astype(jnp.int32), loop_idx, pad_row]
    ).reshape(NW, NCHK, CH)
    NGRP = NCHK // G
    cwr5 = cwr.reshape(2, NW, NGRP, G, CH)
    rowr5 = rowr.reshape(NW, NGRP, G, CH)
    kk = jnp.arange(D // 2)
    lo_col = (kk // 16) * 32 + kk % 16
    xbits = lax.bitcast_convert_type(x.astype(jnp.bfloat16), jnp.uint16)
    xb = (xbits[:, lo_col].astype(jnp.int32)
          | (xbits[:, lo_col + 16].astype(jnp.int32) << 16))

    mesh = plsc.VectorSubcoreMesh(core_axis_name="c", subcore_axis_name="s")
    cp = pltpu.CompilerParams()
    if "needs_layout_passes" in pltpu.CompilerParams.__dataclass_fields__:
        cp = dataclasses.replace(cp, needs_layout_passes=False)
    cpc = dataclasses.replace(cp, use_tc_tiling_on_sc=False)

    @functools.partial(
        pl.kernel,
        out_type=jax.ShapeDtypeStruct((NC * Npad,), jnp.float32),
        mesh=mesh,
        compiler_params=cp,
        scratch_types=[
            pltpu.VMEM((NCHK, CH), jnp.int32),
            pltpu.VMEM((NCHK, CH), jnp.int32),
            pltpu.VMEM((NCHK, CH), jnp.float32),
            pltpu.VMEM((STRIPE,), jnp.float32),
            pltpu.VMEM_SHARED((Npad,), jnp.float32),
            pltpu.SemaphoreType.DMA,
        ],
    )
    def _deg(cw_hbm, degp_hbm, colbuf, ewibuf, ewbuf, zbuf, dshared, sem):
        cid = lax.axis_index("c")
        sid = lax.axis_index("s")
        wid = cid * NS + sid

        ch = pltpu.async_copy(cw_hbm.at[0, wid], colbuf, sem)
        eh = pltpu.async_copy(cw_hbm.at[1, wid], ewibuf, sem)

        @pl.loop(0, STRIPE, step=LN)
        def _(i):
            zbuf[pl.ds(i, LN)] = jnp.zeros((LN,), jnp.float32)

        pltpu.sync_copy(zbuf, dshared.at[pl.ds(sid * STRIPE, STRIPE)])
        ch.wait()
        eh.wait()

        @pl.loop(0, NCHK)
        def _(j):
            for g in range(CH // LN):
                sl = pl.ds(g * LN, LN)
                v = plsc.bitcast(ewibuf[j, sl], jnp.float32)
                ewbuf[j, sl] = jnp.where(v <= jnp.float32(0.0),
                                         jnp.float32(1e-7), v)

        plsc.subcore_barrier()

        @pl.loop(0, NCHK)
        def _(j):
            pltpu.sync_copy(ewbuf.at[j], dshared.at[colbuf.at[j]], add=True)

        plsc.subcore_barrier()

        @pl.when(sid == 0)
        def _():
            pltpu.sync_copy(dshared, degp_hbm.at[pl.ds(cid * Npad, Npad)])

    degp = _deg(cwr)

    @functools.partial(
        pl.kernel,
        out_type=jax.ShapeDtypeStruct((NC * Npad, D), jnp.float32),
        mesh=mesh,
        compiler_params=cpc,
        scratch_types=[
            pltpu.VMEM((G, CH), jnp.int32),
            pltpu.VMEM((2, G, CH), jnp.int32),
            pltpu.VMEM((2, CH), jnp.float32),
            pltpu.VMEM((Npad,), jnp.float32),
            pltpu.VMEM((STRIPE,), jnp.float32),
            pltpu.VMEM((2, CH, D // 2), jnp.int32),
            pltpu.VMEM((CH, D), jnp.float32),
            pltpu.VMEM_SHARED((Npad,), jnp.float32),
            pltpu.VMEM_SHARED((Npad, D), jnp.float32),
            pltpu.SemaphoreType.DMA,
            pltpu.SemaphoreType.DMA,
            pltpu.SemaphoreType.DMA,
        ],
    )
    def _prop(x_hbm, row_hbm, cw_hbm, degp_hbm, accp_hbm,
              rowg, cwall, normbuf, dinvbuf, d1buf, xbf, xf32,
              dinvsh, acc, semg, sems, semi):
        cid = lax.axis_index("c")
        sid = lax.axis_index("s")
        wid = cid * NS + sid

        @pl.loop(0, CH)
        def _(e):
            for q in range(D // LN):
                xf32[e, pl.ds(q * LN, LN)] = jnp.zeros((LN,), jnp.float32)

        @pl.loop(0, STRIPE // CH)
        def _(z):
            pltpu.sync_copy(xf32,
                            acc.at[pl.ds(sid * STRIPE + z * CH, CH)])

        pltpu.sync_copy(degp_hbm.at[pl.ds(sid * STRIPE, STRIPE)],
                        dinvbuf.at[pl.ds(sid * STRIPE, STRIPE)])
        pltpu.sync_copy(degp_hbm.at[pl.ds(Npad + sid * STRIPE, STRIPE)],
                        d1buf)

        @pl.loop(0, STRIPE, step=LN)
        def _(i):
            slg = pl.ds(sid * STRIPE + i, LN)
            sl = pl.ds(i, LN)
            dinvbuf[slg] = _fast_rsqrt(dinvbuf[slg] + d1buf[sl])

        pltpu.sync_copy(dinvbuf.at[pl.ds(sid * STRIPE, STRIPE)],
                        dinvsh.at[pl.ds(sid * STRIPE, STRIPE)])
        plsc.subcore_barrier()
        pltpu.sync_copy(dinvsh, dinvbuf)

        def _norm(q, slot):
            for g in range(CH // LN):
                sl = pl.ds(g * LN, LN)
                ev = plsc.bitcast(cwall[1, q, sl], jnp.float32)
                ev = jnp.where(ev <= jnp.float32(0.0), jnp.float32(1e-7), ev)
                dr = plsc.load_gather(dinvbuf, [rowg[q, sl]])
                dc = plsc.load_gather(dinvbuf, [cwall[0, q, sl]])
                normbuf[slot, sl] = dr * ev * dc

        def _scale(slot):
            @pl.loop(0, CH, step=LN)
            def _(e0):
                nv = normbuf[slot, pl.ds(e0, LN)]
                for l in range(LN):
                    s = nv[l]
                    e = e0 + l
                    for c in range(D // 32):
                        w = xbf[slot, e, pl.ds(c * LN, LN)]
                        lo = plsc.bitcast(w << 16, jnp.float32)
                        hi = plsc.bitcast(w & jnp.int32(-65536), jnp.float32)
                        xf32[e, pl.ds(c * 32, LN)] = lo * s
                        xf32[e, pl.ds(c * 32 + LN, LN)] = hi * s

        @pl.loop(0, NGRP)
        def _(gg):
            h1 = pltpu.async_copy(row_hbm.at[wid, gg], rowg, semi)
            h2 = pltpu.async_copy(cw_hbm.at[pl.ds(0, 2), wid, gg], cwall,
                                  semi)
            h1.wait()
            h2.wait()

            gh = pltpu.async_copy(x_hbm.at[rowg.at[0]], xbf.at[0], semg)
            sh = None
            for q in range(G):
                slot = q % 2
                _norm(q, slot)
                if q + 1 < G:
                    gh2 = pltpu.async_copy(x_hbm.at[rowg.at[q + 1]],
                                           xbf.at[1 - slot], semg)
                gh.wait()
                if sh is not None:
                    sh.wait()
                _scale(slot)
                sh = pltpu.async_copy(xf32, acc.at[cwall.at[0, q]], sems,
                                      add=True)
                if q + 1 < G:
                    gh = gh2
            sh.wait()

        plsc.subcore_barrier()
        pltpu.sync_copy(acc.at[pl.ds(sid * STRIPE, STRIPE)],
                        accp_hbm.at[pl.ds(cid * Npad + sid * STRIPE, STRIPE)])

    accp = _prop(xb, rowr5, cwr5, degp)

    BM = 1024
    NBLK = Npad // BM

    def _mm(h0_ref, h1_ref, w_ref, b_ref, o_ref):
        h = h0_ref[...] + h1_ref[...]
        o_ref[...] = lax.dot_general(
            h, w_ref[...], (((1,), (1,)), ((), ())),
            preferred_element_type=jnp.float32) + b_ref[...]

    out = pl.pallas_call(
        _mm,
        grid=((N + BM - 1) // BM,),
        in_specs=[
            pl.BlockSpec((BM, D), lambda i: (i, 0)),
            pl.BlockSpec((BM, D), lambda i: (i + NBLK, 0)),
            pl.BlockSpec((D, D), lambda i: (0, 0)),
            pl.BlockSpec((1, D), lambda i: (0, 0)),
        ],
        out_specs=pl.BlockSpec((BM, D), lambda i: (i, 0)),
        out_shape=jax.ShapeDtypeStruct((N, D), jnp.float32),
    )(accp, accp, W.astype(jnp.float32), b.reshape(1, D))

    return out


# speedup vs baseline: 1.8238x; 1.8238x over previous
import dataclasses
import functools

import jax
import jax.numpy as jnp
from jax import lax
from jax.experimental import pallas as pl
from jax.experimental.pallas import tpu as pltpu
from jax.experimental.pallas import tpu_sc as plsc

NC = 2
NS = 16
LN = 16
CH = 128


def _fast_rsqrt(d):
    ii = plsc.bitcast(d, jnp.int32)
    ii = jnp.int32(0x5F3759DF) - lax.shift_right_arithmetic(ii, 1)
    y = plsc.bitcast(ii, jnp.float32)
    half_d = d * jnp.float32(0.5)
    for _ in range(3):
        y = y * (jnp.float32(1.5) - half_d * y * y)
    return jnp.where(d > jnp.float32(0.0), y, jnp.float32(0.0))


def kernel(x, edge_index, edge_weights, W, b):
    N, D = x.shape
    E = edge_index.shape[1]
    NW = NC * NS
    npad_unit = NS * CH
    Npad = ((N + 1 + npad_unit - 1) // npad_unit) * npad_unit
    STRIPE = Npad // NS
    G = 8
    EPW = ((E + N + NW * G * CH - 1) // (NW * G * CH)) * (G * CH)
    NCHK = EPW // CH
    PADE = EPW * NW - (E + N)

    loop_idx = jnp.arange(N, dtype=jnp.int32)
    pad_ar = jnp.arange(PADE, dtype=jnp.int32)
    pad_row = pad_ar % N
    pad_col = N + (pad_ar * 7) % (Npad - N)
    one_b = jnp.int32(0x3F800000)
    cwr = jnp.concatenate([
        edge_index[1].astype(jnp.int32), loop_idx, pad_col,
        lax.bitcast_convert_type(edge_weights.astype(jnp.float32),
                                 jnp.int32),
        jnp.full((N,), one_b, jnp.int32), jnp.zeros((PADE,), jnp.int32),
    ]).reshape(2, NW, NCHK, CH)
    rowr = jnp.concatenate(
        [edge_index[0].astype(jnp.int32), loop_idx, pad_row]
    ).reshape(NW, NCHK, CH)
    NGRP = NCHK // G
    cwr5 = cwr.reshape(2, NW, NGRP, G, CH)
    rowr5 = rowr.reshape(NW, NGRP, G, CH)
    xp = x.astype(jnp.float32)

    mesh = plsc.VectorSubcoreMesh(core_axis_name="c", subcore_axis_name="s")
    cp = pltpu.CompilerParams()
    if "needs_layout_passes" in pltpu.CompilerParams.__dataclass_fields__:
        cp = dataclasses.replace(cp, needs_layout_passes=False)

    @functools.partial(
        pl.kernel,
        out_type=jax.ShapeDtypeStruct((NC * Npad,), jnp.float32),
        mesh=mesh,
        compiler_params=cp,
        scratch_types=[
            pltpu.VMEM((NCHK, CH), jnp.int32),
            pltpu.VMEM((NCHK, CH), jnp.int32),
            pltpu.VMEM((NCHK, CH), jnp.float32),
            pltpu.VMEM((STRIPE,), jnp.float32),
            pltpu.VMEM_SHARED((Npad,), jnp.float32),
            pltpu.SemaphoreType.DMA,
        ],
    )
    def _deg(cw_hbm, degp_hbm, colbuf, ewibuf, ewbuf, zbuf, dshared, sem):
        cid = lax.axis_index("c")
        sid = lax.axis_index("s")
        wid = cid * NS + sid

        ch = pltpu.async_copy(cw_hbm.at[0, wid], colbuf, sem)
        eh = pltpu.async_copy(cw_hbm.at[1, wid], ewibuf, sem)

        @pl.loop(0, STRIPE, step=LN)
        def _(i):
            zbuf[pl.ds(i, LN)] = jnp.zeros((LN,), jnp.float32)

        pltpu.sync_copy(zbuf, dshared.at[pl.ds(sid * STRIPE, STRIPE)])
        ch.wait()
        eh.wait()

        @pl.loop(0, NCHK)
        def _(j):
            for g in range(CH // LN):
                sl = pl.ds(g * LN, LN)
                v = plsc.bitcast(ewibuf[j, sl], jnp.float32)
                ewbuf[j, sl] = jnp.where(v <= jnp.float32(0.0),
                                         jnp.float32(1e-7), v)

        plsc.subcore_barrier()

        @pl.loop(0, NCHK)
        def _(j):
            pltpu.sync_copy(ewbuf.at[j], dshared.at[colbuf.at[j]], add=True)

        plsc.subcore_barrier()

        @pl.when(sid == 0)
        def _():
            pltpu.sync_copy(dshared, degp_hbm.at[pl.ds(cid * Npad, Npad)])

    degp = _deg(cwr)

    @functools.partial(
        pl.kernel,
        out_type=jax.ShapeDtypeStruct((NC * Npad, D), jnp.float32),
        mesh=mesh,
        compiler_params=cp,
        scratch_types=[
            pltpu.VMEM((G, CH), jnp.int32),
            pltpu.VMEM((2, G, CH), jnp.int32),
            pltpu.VMEM((2, CH), jnp.float32),
            pltpu.VMEM((Npad,), jnp.float32),
            pltpu.VMEM((STRIPE,), jnp.float32),
            pltpu.VMEM((2, CH, D), jnp.float32),
            pltpu.VMEM_SHARED((Npad,), jnp.float32),
            pltpu.VMEM_SHARED((Npad, D), jnp.float32),
            pltpu.SemaphoreType.DMA,
            pltpu.SemaphoreType.DMA,
            pltpu.SemaphoreType.DMA,
        ],
    )
    def _prop(x_hbm, row_hbm, cw_hbm, degp_hbm, accp_hbm,
              rowg, cwall, normbuf, dinvbuf, d1buf, xrows,
              dinvsh, acc, semg, sems, semi):
        cid = lax.axis_index("c")
        sid = lax.axis_index("s")
        wid = cid * NS + sid

        @pl.loop(0, CH)
        def _(e):
            for q in range(D // LN):
                xrows[0, e, pl.ds(q * LN, LN)] = jnp.zeros((LN,), jnp.float32)

        @pl.loop(0, STRIPE // CH)
        def _(z):
            pltpu.sync_copy(xrows.at[0],
                            acc.at[pl.ds(sid * STRIPE + z * CH, CH)])

        pltpu.sync_copy(degp_hbm.at[pl.ds(sid * STRIPE, STRIPE)],
                        dinvbuf.at[pl.ds(sid * STRIPE, STRIPE)])
        pltpu.sync_copy(degp_hbm.at[pl.ds(Npad + sid * STRIPE, STRIPE)],
                        d1buf)

        @pl.loop(0, STRIPE, step=LN)
        def _(i):
            slg = pl.ds(sid * STRIPE + i, LN)
            sl = pl.ds(i, LN)
            dinvbuf[slg] = _fast_rsqrt(dinvbuf[slg] + d1buf[sl])

        pltpu.sync_copy(dinvbuf.at[pl.ds(sid * STRIPE, STRIPE)],
                        dinvsh.at[pl.ds(sid * STRIPE, STRIPE)])
        plsc.subcore_barrier()
        pltpu.sync_copy(dinvsh, dinvbuf)

        def _norm(q, slot):
            for g in range(CH // LN):
                sl = pl.ds(g * LN, LN)
                ev = plsc.bitcast(cwall[1, q, sl], jnp.float32)
                ev = jnp.where(ev <= jnp.float32(0.0), jnp.float32(1e-7), ev)
                dr = plsc.load_gather(dinvbuf, [rowg[q, sl]])
                dc = plsc.load_gather(dinvbuf, [cwall[0, q, sl]])
                normbuf[slot, sl] = dr * ev * dc

        def _scale(slot):
            @pl.loop(0, CH, step=LN)
            def _(e0):
                nv = normbuf[slot, pl.ds(e0, LN)]
                for l in range(LN):
                    s = nv[l]
                    for q in range(D // LN):
                        sl = pl.ds(q * LN, LN)
                        xrows[slot, e0 + l, sl] = xrows[slot, e0 + l, sl] * s

        @pl.loop(0, NGRP)
        def _(gg):
            h1 = pltpu.async_copy(row_hbm.at[wid, gg], rowg, semi)
            h2 = pltpu.async_copy(cw_hbm.at[pl.ds(0, 2), wid, gg], cwall,
                                  semi)
            h1.wait()
            h2.wait()

            gh = pltpu.async_copy(x_hbm.at[rowg.at[0]], xrows.at[0], semg)
            sh = [None, None]
            for q in range(G):
                slot = q % 2
                _norm(q, slot)
                if sh[1 - slot] is not None:
                    sh[1 - slot].wait()
                if q + 1 < G:
                    gh2 = pltpu.async_copy(x_hbm.at[rowg.at[q + 1]],
                                           xrows.at[1 - slot], semg)
                gh.wait()
                _scale(slot)
                sh[slot] = pltpu.async_copy(xrows.at[slot],
                                            acc.at[cwall.at[0, q]], sems,
                                            add=True)
                if q + 1 < G:
                    gh = gh2
            sh[(G - 1) % 2].wait()

        plsc.subcore_barrier()
        pltpu.sync_copy(acc.at[pl.ds(sid * STRIPE, STRIPE)],
                        accp_hbm.at[pl.ds(cid * Npad + sid * STRIPE, STRIPE)])

    accp = _prop(xp, rowr5, cwr5, degp)

    BM = 1024
    NBLK = Npad // BM

    def _mm(h0_ref, h1_ref, w_ref, b_ref, o_ref):
        h = h0_ref[...] + h1_ref[...]
        o_ref[...] = lax.dot_general(
            h, w_ref[...], (((1,), (1,)), ((), ())),
            preferred_element_type=jnp.float32) + b_ref[...]

    out = pl.pallas_call(
        _mm,
        grid=((N + BM - 1) // BM,),
        in_specs=[
            pl.BlockSpec((BM, D), lambda i: (i, 0)),
            pl.BlockSpec((BM, D), lambda i: (i + NBLK, 0)),
            pl.BlockSpec((D, D), lambda i: (0, 0)),
            pl.BlockSpec((1, D), lambda i: (0, 0)),
        ],
        out_specs=pl.BlockSpec((BM, D), lambda i: (i, 0)),
        out_shape=jax.ShapeDtypeStruct((N, D), jnp.float32),
    )(accp, accp, W.astype(jnp.float32), b.reshape(1, D))

    return out
